# baseline (speedup 1.0000x reference)
import functools

import jax
import jax.numpy as jnp
import numpy as np
from jax import lax
from jax.experimental import pallas as pl
from jax.experimental.pallas import tpu as pltpu
from jax.experimental.pallas import tpu_sc as plsc

VOCAB = 100000
MAX_SEQ_LEN = 2048
D_MODEL = 768

B = 4
S = 2048
NW = 32
S_PER_W = S // NW
LANES = 16
VECS_PER_ROW = D_MODEL // LANES


def _pos_encoding(max_seq_len, d_model):
    pos = np.arange(max_seq_len, dtype=np.float32)[:, None]
    div = np.exp(
        np.arange(0, d_model, 2, dtype=np.float32) * (-np.log(10000.0) / d_model)
    )
    pe = np.zeros((max_seq_len, d_model), dtype=np.float32)
    pe[:, 0::2] = np.sin(pos * div)
    pe[:, 1::2] = np.cos(pos * div)
    return pe


_PE = _pos_encoding(MAX_SEQ_LEN, D_MODEL)


CH = 16
NCH = (B * S_PER_W) // CH
NBUF = 6
CHUNKS_PER_B = S_PER_W // CH


def _make_sc_call():
    mesh = plsc.VectorSubcoreMesh(core_axis_name="c", subcore_axis_name="s")

    @functools.partial(
        pl.kernel,
        mesh=mesh,
        out_type=jax.ShapeDtypeStruct((B, S, D_MODEL), jnp.float32),
        scratch_types=[
            pltpu.VMEM((B, S_PER_W), jnp.int32),
            pltpu.VMEM((S_PER_W, D_MODEL), jnp.float32),
        ]
        + [pltpu.VMEM((CH, D_MODEL), jnp.float32) for _ in range(NBUF)]
        + [pltpu.SemaphoreType.DMA for _ in range(2 * NBUF + 1)],
    )
    def emb_kernel(xt_hbm, w_hbm, pe_hbm, out_hbm, idx_v, pe_v, *bufs_and_sems):
        rows = bufs_and_sems[:NBUF]
        gsem = bufs_and_sems[NBUF:2 * NBUF]
        osem = bufs_and_sems[2 * NBUF:3 * NBUF]
        psem = bufs_and_sems[3 * NBUF]
        wid = lax.axis_index("s") * 2 + lax.axis_index("c")
        seq_base = wid * S_PER_W

        pe_copy = pltpu.async_copy(pe_hbm.at[wid], pe_v, psem)
        pltpu.sync_copy(xt_hbm.at[wid], idx_v)

        def fire_gather(c):
            b, h = divmod(c, CHUNKS_PER_B)
            s = c % NBUF
            return pltpu.async_copy(
                w_hbm.at[idx_v.at[b, pl.ds(h * CH, CH)]], rows[s], gsem[s])

        gathers = {}
        outs = {}
        for c in range(min(NBUF, NCH)):
            gathers[c] = fire_gather(c)
        pe_copy.wait()

        for c in range(NCH):
            b, h = divmod(c, CHUNKS_PER_B)
            s = c % NBUF
            gathers[c].wait()

            nxt = c + NBUF - 2
            if NBUF <= nxt < NCH:
                outs[nxt - NBUF].wait()
                gathers[nxt] = fire_gather(nxt)

            def add_row(i, _, _s=s, _h=h):
                for j in range(VECS_PER_ROW):
                    sl = pl.ds(j * LANES, LANES)
                    plsc.addupdate(rows[_s].at[i, sl], pe_v[_h * CH + i, sl])
                return 0

            lax.fori_loop(0, CH, add_row, 0)

            outs[c] = pltpu.async_copy(
                rows[s], out_hbm.at[b, pl.ds(seq_base + h * CH, CH)], osem[s])

        for c in range(max(0, NCH - NBUF), NCH):
            outs[c].wait()

    return emb_kernel


_SC_CALL = _make_sc_call()


def kernel(x, W):
    xt = x.astype(jnp.int32).reshape(B, NW, S_PER_W).transpose(1, 0, 2)
    pe = jnp.asarray(_PE).reshape(NW, S_PER_W, D_MODEL)
    return _SC_CALL(xt, W, pe)


# speedup vs baseline: 1.2536x; 1.2536x over previous
import functools

import jax
import jax.numpy as jnp
import numpy as np
from jax import lax
from jax.experimental import pallas as pl
from jax.experimental.pallas import tpu as pltpu
from jax.experimental.pallas import tpu_sc as plsc

VOCAB = 100000
MAX_SEQ_LEN = 2048
D_MODEL = 768

B = 4
S = 2048
NW = 32
S_PER_W = S // NW
LANES = 16
VECS_PER_ROW = D_MODEL // LANES

CH = 8
NGRP = S_PER_W // CH
RING = 3
NBUF = RING * B


def _pos_encoding(max_seq_len, d_model):
    pos = np.arange(max_seq_len, dtype=np.float32)[:, None]
    div = np.exp(
        np.arange(0, d_model, 2, dtype=np.float32) * (-np.log(10000.0) / d_model)
    )
    pe = np.zeros((max_seq_len, d_model), dtype=np.float32)
    pe[:, 0::2] = np.sin(pos * div)
    pe[:, 1::2] = np.cos(pos * div)
    return pe


_PE = _pos_encoding(MAX_SEQ_LEN, D_MODEL)


def _make_sc_call():
    mesh = plsc.VectorSubcoreMesh(core_axis_name="c", subcore_axis_name="s")

    @functools.partial(
        pl.kernel,
        mesh=mesh,
        out_type=jax.ShapeDtypeStruct((B, S, D_MODEL), jnp.float32),
        scratch_types=[
            pltpu.VMEM((B, S_PER_W), jnp.int32),
            pltpu.VMEM((S_PER_W, D_MODEL), jnp.float32),
        ]
        + [pltpu.VMEM((CH, D_MODEL), jnp.float32) for _ in range(NBUF)]
        + [pltpu.SemaphoreType.DMA for _ in range(2 * NBUF + 1)],
    )
    def emb_kernel(x_hbm, w_hbm, pe_hbm, out_hbm, idx_v, pe_v, *bufs_and_sems):
        rows = bufs_and_sems[:NBUF]
        gsem = bufs_and_sems[NBUF:2 * NBUF]
        osem = bufs_and_sems[2 * NBUF:3 * NBUF]
        psem = bufs_and_sems[3 * NBUF]
        wid = lax.axis_index("s") * 2 + lax.axis_index("c")
        seq_base = wid * S_PER_W

        pe_copy = pltpu.async_copy(pe_hbm.at[wid], pe_v, psem)
        for b in range(B):
            pltpu.sync_copy(x_hbm.at[b, pl.ds(seq_base, S_PER_W)], idx_v.at[b])

        def slot(h, b):
            return (h % RING) * B + b

        def fire_gather(h, b):
            s = slot(h, b)
            return pltpu.async_copy(
                w_hbm.at[idx_v.at[b, pl.ds(h * CH, CH)]], rows[s], gsem[s])

        def fire_out(h, b):
            s = slot(h, b)
            return pltpu.async_copy(
                rows[s], out_hbm.at[b, pl.ds(seq_base + h * CH, CH)], osem[s])

        gathers = {}
        outs = {}
        for h in range(min(RING - 1, NGRP)):
            for b in range(B):
                gathers[(h, b)] = fire_gather(h, b)
        pe_copy.wait()

        for h in range(NGRP):
            nh = h + RING - 1
            if nh < NGRP:
                if nh >= RING:
                    for b in range(B):
                        outs[(nh - RING, b)].wait()
                for b in range(B):
                    gathers[(nh, b)] = fire_gather(nh, b)
            for b in range(B):
                gathers[(h, b)].wait()

            bslots = [slot(h, b) for b in range(B)]

            def add_row(i, _, _h=h, _bslots=bslots):
                for j in range(VECS_PER_ROW):
                    sl = pl.ds(j * LANES, LANES)
                    pv = pe_v[_h * CH + i, sl]
                    for s in _bslots:
                        plsc.addupdate(rows[s].at[i, sl], pv)
                return 0

            lax.fori_loop(0, CH, add_row, 0)
            for b in range(B):
                outs[(h, b)] = fire_out(h, b)

        for h in range(max(0, NGRP - RING), NGRP):
            for b in range(B):
                outs[(h, b)].wait()

    return emb_kernel


_SC_CALL = _make_sc_call()


def kernel(x, W):
    xt = x.astype(jnp.int32)
    pe = jnp.asarray(_PE).reshape(NW, S_PER_W, D_MODEL)
    return _SC_CALL(xt, W, pe)
